# baseline (speedup 1.0000x reference)
import jax
import jax.numpy as jnp
from jax import lax
from jax.experimental import pallas as pl
from jax.experimental.pallas import tpu as pltpu

B, C, H, W = 8, 64, 128, 128
HW = H * W
P0 = W
HWQ = HW + 2 * P0
CHW = 2048
NCH = HW // CHW


def _mk_shifts(s0, sm, sp):
    v = s0[...]
    cin = v.shape[1]
    q = lax.broadcasted_iota(jnp.int32, (HWQ, cin), 0)
    w_of_q = q & (W - 1)
    sm[...] = jnp.where((w_of_q != 0) & (q < P0 + HW),
                        pltpu.roll(v, 1, 0), jnp.bfloat16(0))
    sp[...] = jnp.where((w_of_q != W - 1) & (q >= P0),
                        pltpu.roll(v, HWQ - 1, 0), jnp.bfloat16(0))


def _conv_chunk(srcs, r0, w_ref, b_ref, relu=True):
    taps = []
    for s in srcs:
        for dh in (-1, 0, 1):
            taps.append(s[pl.ds(P0 + r0 + dh * W, CHW), :])
    z = jnp.concatenate(taps, axis=1)
    acc = lax.dot(z, w_ref[...], preferred_element_type=jnp.float32)
    acc = acc + b_ref[...]
    if relu:
        acc = jnp.maximum(acc, 0.0)
    return acc


def _head_kernel(x_ref, w1_ref, b1_ref, cw2_ref, cb2_ref, cwo_ref, cbo_ref,
                 ww2_ref, wb2_ref, wwo_ref, wbo_ref,
                 cls_ref, wh_ref,
                 xin, xs0, xsm, xsp, hs0, hsm, hsp, gs0, dsem):
    gsm, gsp = xsm, xsp
    b = pl.program_id(0)
    cp = pltpu.make_async_copy(x_ref.at[b], xin, dsem)
    cp.start()

    @pl.when(b == 0)
    def _zero_pads():
        for ref, cc in ((xs0, C), (hs0, 128), (gs0, C)):
            ref[0:P0, :] = jnp.zeros((P0, cc), jnp.bfloat16)
            ref[P0 + HW:, :] = jnp.zeros((P0, cc), jnp.bfloat16)

    cp.wait()
    for ci in range(NCH):
        r0 = ci * CHW
        xs0[pl.ds(P0 + r0, CHW), :] = jnp.transpose(
            xin[:, pl.ds(r0, CHW)], (1, 0)).astype(jnp.bfloat16)
    _mk_shifts(xs0, xsm, xsp)

    for ci in range(NCH):
        r0 = ci * CHW
        a = _conv_chunk((xsm, xs0, xsp), r0, w1_ref, b1_ref)
        ab = a.astype(jnp.bfloat16)
        hs0[pl.ds(P0 + r0, CHW), :] = ab[:, :128]
        gs0[pl.ds(P0 + r0, CHW), :] = ab[:, 128:]

    _mk_shifts(hs0, hsm, hsp)
    _mk_shifts(gs0, gsm, gsp)

    for ci in range(NCH):
        r0 = ci * CHW
        h2 = _conv_chunk((hsm, hs0, hsp), r0, cw2_ref, cb2_ref)
        cls = lax.dot(h2.astype(jnp.bfloat16), cwo_ref[...],
                      preferred_element_type=jnp.float32) + cbo_ref[...]
        cls_ref[0, :, pl.ds(r0, CHW)] = jnp.transpose(cls, (1, 0))

        g2 = _conv_chunk((gsm, gs0, gsp), r0, ww2_ref, wb2_ref)
        wh = lax.dot(g2.astype(jnp.bfloat16), wwo_ref[...],
                     preferred_element_type=jnp.float32) + wbo_ref[...]
        wh = jnp.maximum(wh, 0.0) * 16.0
        wh_ref[0, :, pl.ds(r0, CHW)] = jnp.transpose(wh, (1, 0))[:4, :]


def _prep_w3(w):
    wt = jnp.transpose(w, (3, 2, 1, 0))
    return wt.reshape(9 * w.shape[1], w.shape[0]).astype(jnp.bfloat16)


def kernel(x, cls_w0, cls_b0, cls_w1, cls_b1, cls_wout, cls_bout,
           wh_w0, wh_b0, wh_w1, wh_b1, wh_wout, wh_bout):
    xt = x.reshape(B, C, HW)

    w1 = jnp.concatenate([_prep_w3(cls_w0), _prep_w3(wh_w0)], axis=1)
    b1 = jnp.concatenate([cls_b0, wh_b0]).reshape(1, 192)
    cw2 = _prep_w3(cls_w1)
    ww2 = _prep_w3(wh_w1)
    cwo = jnp.transpose(cls_wout[:, :, 0, 0], (1, 0)).astype(jnp.bfloat16)
    wwo = jnp.transpose(wh_wout[:, :, 0, 0], (1, 0)).astype(jnp.bfloat16)
    wwo = jnp.pad(wwo, ((0, 0), (0, 4)))

    cb2 = cls_b1.reshape(1, 128)
    cbo = cls_bout.reshape(1, 80)
    wb2 = wh_b1.reshape(1, 64)
    wbo = jnp.pad(wh_bout, (0, 4)).reshape(1, 8)

    fixed = lambda *shape: pl.BlockSpec(shape, lambda b: (0,) * len(shape))
    cls_t, wh_t = pl.pallas_call(
        _head_kernel,
        grid=(B,),
        in_specs=[
            pl.BlockSpec(memory_space=pltpu.MemorySpace.HBM),
            fixed(9 * C, 192), fixed(1, 192),
            fixed(9 * 128, 128), fixed(1, 128),
            fixed(128, 80), fixed(1, 80),
            fixed(9 * C, 64), fixed(1, 64),
            fixed(C, 8), fixed(1, 8),
        ],
        out_specs=[
            pl.BlockSpec((1, 80, HW), lambda b: (b, 0, 0)),
            pl.BlockSpec((1, 4, HW), lambda b: (b, 0, 0)),
        ],
        out_shape=[
            jax.ShapeDtypeStruct((B, 80, HW), jnp.float32),
            jax.ShapeDtypeStruct((B, 4, HW), jnp.float32),
        ],
        scratch_shapes=[
            pltpu.VMEM((C, HW), jnp.float32),
            pltpu.VMEM((HWQ, C), jnp.bfloat16),
            pltpu.VMEM((HWQ, C), jnp.bfloat16),
            pltpu.VMEM((HWQ, C), jnp.bfloat16),
            pltpu.VMEM((HWQ, 128), jnp.bfloat16),
            pltpu.VMEM((HWQ, 128), jnp.bfloat16),
            pltpu.VMEM((HWQ, 128), jnp.bfloat16),
            pltpu.VMEM((HWQ, C), jnp.bfloat16),
            pltpu.SemaphoreType.DMA,
        ],
        compiler_params=pltpu.CompilerParams(
            dimension_semantics=("arbitrary",),
            vmem_limit_bytes=64 * 1024 * 1024,
        ),
    )(xt, w1, b1, cw2, cb2, cwo, cbo, ww2, wb2, wwo, wbo)

    cls = cls_t.reshape(B, 80, H, W)
    wh = wh_t.reshape(B, 4, H, W)
    return (cls, wh)


# speedup vs baseline: 1.0407x; 1.0407x over previous
import jax
import jax.numpy as jnp
from jax import lax
from jax.experimental import pallas as pl
from jax.experimental.pallas import tpu as pltpu

B, C, H, W = 8, 64, 128, 128
HW = H * W
P0 = W
HWQ = HW + 2 * P0
CHW = 2048
NCH = HW // CHW


def _mk_shifts(s0, sm, sp):
    v = s0[...]
    cin = v.shape[1]
    q = lax.broadcasted_iota(jnp.int32, (HWQ, cin), 0)
    w_of_q = q & (W - 1)
    sm[...] = jnp.where((w_of_q != 0) & (q < P0 + HW),
                        pltpu.roll(v, 1, 0), jnp.bfloat16(0))
    sp[...] = jnp.where((w_of_q != W - 1) & (q >= P0),
                        pltpu.roll(v, HWQ - 1, 0), jnp.bfloat16(0))


def _conv_chunk(srcs, r0, w_ref, b_ref, relu=True):
    taps = []
    for s in srcs:
        for dh in (-1, 0, 1):
            taps.append(s[pl.ds(P0 + r0 + dh * W, CHW), :])
    z = jnp.concatenate(taps, axis=1)
    acc = lax.dot(z, w_ref[...], preferred_element_type=jnp.float32)
    acc = acc + b_ref[...]
    if relu:
        acc = jnp.maximum(acc, 0.0)
    return acc


def _head_kernel(x_ref, w1_ref, b1_ref, cw2_ref, cb2_ref, cwo_ref, cbo_ref,
                 ww2_ref, wb2_ref, wwo_ref, wbo_ref,
                 cls_ref, wh_ref,
                 xs0, xsm, xsp, hs0, hsm, hsp, gs0, dsem):
    gsm, gsp = xsm, xsp
    b = pl.program_id(0)
    cp = pltpu.make_async_copy(x_ref.at[b], xs0.at[pl.ds(P0, HW), :], dsem)
    cp.start()

    @pl.when(b == 0)
    def _zero_pads():
        for ref, cc in ((xs0, C), (hs0, 128), (gs0, C)):
            ref[0:P0, :] = jnp.zeros((P0, cc), jnp.bfloat16)
            ref[P0 + HW:, :] = jnp.zeros((P0, cc), jnp.bfloat16)

    cp.wait()
    _mk_shifts(xs0, xsm, xsp)

    for ci in range(NCH):
        r0 = ci * CHW
        a = _conv_chunk((xsm, xs0, xsp), r0, w1_ref, b1_ref)
        ab = a.astype(jnp.bfloat16)
        hs0[pl.ds(P0 + r0, CHW), :] = ab[:, :128]
        gs0[pl.ds(P0 + r0, CHW), :] = ab[:, 128:]

    _mk_shifts(hs0, hsm, hsp)
    _mk_shifts(gs0, gsm, gsp)

    for ci in range(NCH):
        r0 = ci * CHW
        h2 = _conv_chunk((hsm, hs0, hsp), r0, cw2_ref, cb2_ref)
        cls = lax.dot(h2.astype(jnp.bfloat16), cwo_ref[...],
                      preferred_element_type=jnp.float32) + cbo_ref[...]
        cls_ref[0, :, pl.ds(r0, CHW)] = jnp.transpose(cls, (1, 0))

        g2 = _conv_chunk((gsm, gs0, gsp), r0, ww2_ref, wb2_ref)
        wh = lax.dot(g2.astype(jnp.bfloat16), wwo_ref[...],
                     preferred_element_type=jnp.float32) + wbo_ref[...]
        wh = jnp.maximum(wh, 0.0) * 16.0
        wh_ref[0, :, pl.ds(r0, CHW)] = jnp.transpose(wh, (1, 0))[:4, :]


def _prep_w3(w):
    wt = jnp.transpose(w, (3, 2, 1, 0))
    return wt.reshape(9 * w.shape[1], w.shape[0]).astype(jnp.bfloat16)


def kernel(x, cls_w0, cls_b0, cls_w1, cls_b1, cls_wout, cls_bout,
           wh_w0, wh_b0, wh_w1, wh_b1, wh_wout, wh_bout):
    xt = jnp.transpose(x, (0, 2, 3, 1)).reshape(B, HW, C).astype(jnp.bfloat16)

    w1 = jnp.concatenate([_prep_w3(cls_w0), _prep_w3(wh_w0)], axis=1)
    b1 = jnp.concatenate([cls_b0, wh_b0]).reshape(1, 192)
    cw2 = _prep_w3(cls_w1)
    ww2 = _prep_w3(wh_w1)
    cwo = jnp.transpose(cls_wout[:, :, 0, 0], (1, 0)).astype(jnp.bfloat16)
    wwo = jnp.transpose(wh_wout[:, :, 0, 0], (1, 0)).astype(jnp.bfloat16)
    wwo = jnp.pad(wwo, ((0, 0), (0, 4)))

    cb2 = cls_b1.reshape(1, 128)
    cbo = cls_bout.reshape(1, 80)
    wb2 = wh_b1.reshape(1, 64)
    wbo = jnp.pad(wh_bout, (0, 4)).reshape(1, 8)

    fixed = lambda *shape: pl.BlockSpec(shape, lambda b: (0,) * len(shape))
    cls_t, wh_t = pl.pallas_call(
        _head_kernel,
        grid=(B,),
        in_specs=[
            pl.BlockSpec(memory_space=pltpu.MemorySpace.HBM),
            fixed(9 * C, 192), fixed(1, 192),
            fixed(9 * 128, 128), fixed(1, 128),
            fixed(128, 80), fixed(1, 80),
            fixed(9 * C, 64), fixed(1, 64),
            fixed(C, 8), fixed(1, 8),
        ],
        out_specs=[
            pl.BlockSpec((1, 80, HW), lambda b: (b, 0, 0)),
            pl.BlockSpec((1, 4, HW), lambda b: (b, 0, 0)),
        ],
        out_shape=[
            jax.ShapeDtypeStruct((B, 80, HW), jnp.float32),
            jax.ShapeDtypeStruct((B, 4, HW), jnp.float32),
        ],
        scratch_shapes=[
            pltpu.VMEM((HWQ, C), jnp.bfloat16),
            pltpu.VMEM((HWQ, C), jnp.bfloat16),
            pltpu.VMEM((HWQ, C), jnp.bfloat16),
            pltpu.VMEM((HWQ, 128), jnp.bfloat16),
            pltpu.VMEM((HWQ, 128), jnp.bfloat16),
            pltpu.VMEM((HWQ, 128), jnp.bfloat16),
            pltpu.VMEM((HWQ, C), jnp.bfloat16),
            pltpu.SemaphoreType.DMA,
        ],
        compiler_params=pltpu.CompilerParams(
            dimension_semantics=("arbitrary",),
            vmem_limit_bytes=64 * 1024 * 1024,
        ),
    )(xt, w1, b1, cw2, cb2, cwo, cbo, ww2, wb2, wwo, wbo)

    cls = cls_t.reshape(B, 80, H, W)
    wh = wh_t.reshape(B, 4, H, W)
    return (cls, wh)


# speedup vs baseline: 1.1099x; 1.0665x over previous
import jax
import jax.numpy as jnp
from jax import lax
from jax.experimental import pallas as pl
from jax.experimental.pallas import tpu as pltpu

B, C, H, W = 8, 64, 128, 128
HW = H * W
P0 = W
HWQ = HW + 2 * P0
CHW = 1024
NCH = HW // CHW


def _mk_shifts(s0, sm, sp):
    v = s0[...]
    cin = v.shape[1]
    q = lax.broadcasted_iota(jnp.int32, (HWQ, cin), 0)
    w_of_q = q & (W - 1)
    sm[...] = jnp.where((w_of_q != 0) & (q < P0 + HW),
                        pltpu.roll(v, 1, 0), jnp.bfloat16(0))
    sp[...] = jnp.where((w_of_q != W - 1) & (q >= P0),
                        pltpu.roll(v, HWQ - 1, 0), jnp.bfloat16(0))


def _conv_chunk(srcs, r0, w_ref, b_ref, relu=True):
    taps = []
    for s in srcs:
        for dh in (-1, 0, 1):
            taps.append(s[pl.ds(P0 + r0 + dh * W, CHW), :])
    z = jnp.concatenate(taps, axis=1)
    acc = lax.dot(z, w_ref[...], preferred_element_type=jnp.float32)
    acc = acc + b_ref[...]
    if relu:
        acc = jnp.maximum(acc, 0.0)
    return acc


def _head_kernel(x_ref, w1_ref, b1_ref, cw2_ref, cb2_ref, cwo_ref, cbo_ref,
                 ww2_ref, wb2_ref, wwo_ref, wbo_ref,
                 cls_ref, wh_ref,
                 xs0, xsm, xsp, hs0, hsm, hsp, gs0, dsem):
    gsm, gsp = xsm, xsp
    b = pl.program_id(0)
    cp = pltpu.make_async_copy(x_ref.at[b], xs0.at[pl.ds(P0, HW), :], dsem)
    cp.start()

    @pl.when(b == 0)
    def _zero_pads():
        for ref, cc in ((xs0, C), (hs0, 128), (gs0, C)):
            ref[0:P0, :] = jnp.zeros((P0, cc), jnp.bfloat16)
            ref[P0 + HW:, :] = jnp.zeros((P0, cc), jnp.bfloat16)

    cp.wait()
    _mk_shifts(xs0, xsm, xsp)

    for ci in range(NCH):
        r0 = ci * CHW
        a = _conv_chunk((xsm, xs0, xsp), r0, w1_ref, b1_ref)
        ab = a.astype(jnp.bfloat16)
        hs0[pl.ds(P0 + r0, CHW), :] = ab[:, :128]
        gs0[pl.ds(P0 + r0, CHW), :] = ab[:, 128:]

    _mk_shifts(hs0, hsm, hsp)
    _mk_shifts(gs0, gsm, gsp)

    for ci in range(NCH):
        r0 = ci * CHW
        h2 = _conv_chunk((hsm, hs0, hsp), r0, cw2_ref, cb2_ref)
        cls = lax.dot(h2.astype(jnp.bfloat16), cwo_ref[...],
                      preferred_element_type=jnp.float32) + cbo_ref[...]
        cls_ref[0, :, pl.ds(r0, CHW)] = jnp.transpose(cls, (1, 0))

        g2 = _conv_chunk((gsm, gs0, gsp), r0, ww2_ref, wb2_ref)
        wh = lax.dot(g2.astype(jnp.bfloat16), wwo_ref[...],
                     preferred_element_type=jnp.float32) + wbo_ref[...]
        wh = jnp.maximum(wh, 0.0) * 16.0
        wh_ref[0, :, pl.ds(r0, CHW)] = jnp.transpose(wh, (1, 0))[:4, :]


def _prep_w3(w):
    wt = jnp.transpose(w, (3, 2, 1, 0))
    return wt.reshape(9 * w.shape[1], w.shape[0]).astype(jnp.bfloat16)


def kernel(x, cls_w0, cls_b0, cls_w1, cls_b1, cls_wout, cls_bout,
           wh_w0, wh_b0, wh_w1, wh_b1, wh_wout, wh_bout):
    xt = jnp.transpose(x, (0, 2, 3, 1)).reshape(B, HW, C).astype(jnp.bfloat16)

    w1 = jnp.concatenate([_prep_w3(cls_w0), _prep_w3(wh_w0)], axis=1)
    b1 = jnp.concatenate([cls_b0, wh_b0]).reshape(1, 192)
    cw2 = _prep_w3(cls_w1)
    ww2 = _prep_w3(wh_w1)
    cwo = jnp.transpose(cls_wout[:, :, 0, 0], (1, 0)).astype(jnp.bfloat16)
    wwo = jnp.transpose(wh_wout[:, :, 0, 0], (1, 0)).astype(jnp.bfloat16)
    wwo = jnp.pad(wwo, ((0, 0), (0, 4)))

    cb2 = cls_b1.reshape(1, 128)
    cbo = cls_bout.reshape(1, 80)
    wb2 = wh_b1.reshape(1, 64)
    wbo = jnp.pad(wh_bout, (0, 4)).reshape(1, 8)

    fixed = lambda *shape: pl.BlockSpec(shape, lambda b: (0,) * len(shape))
    cls_t, wh_t = pl.pallas_call(
        _head_kernel,
        grid=(B,),
        in_specs=[
            pl.BlockSpec(memory_space=pltpu.MemorySpace.HBM),
            fixed(9 * C, 192), fixed(1, 192),
            fixed(9 * 128, 128), fixed(1, 128),
            fixed(128, 80), fixed(1, 80),
            fixed(9 * C, 64), fixed(1, 64),
            fixed(C, 8), fixed(1, 8),
        ],
        out_specs=[
            pl.BlockSpec((1, 80, HW), lambda b: (b, 0, 0)),
            pl.BlockSpec((1, 4, HW), lambda b: (b, 0, 0)),
        ],
        out_shape=[
            jax.ShapeDtypeStruct((B, 80, HW), jnp.float32),
            jax.ShapeDtypeStruct((B, 4, HW), jnp.float32),
        ],
        scratch_shapes=[
            pltpu.VMEM((HWQ, C), jnp.bfloat16),
            pltpu.VMEM((HWQ, C), jnp.bfloat16),
            pltpu.VMEM((HWQ, C), jnp.bfloat16),
            pltpu.VMEM((HWQ, 128), jnp.bfloat16),
            pltpu.VMEM((HWQ, 128), jnp.bfloat16),
            pltpu.VMEM((HWQ, 128), jnp.bfloat16),
            pltpu.VMEM((HWQ, C), jnp.bfloat16),
            pltpu.SemaphoreType.DMA,
        ],
        compiler_params=pltpu.CompilerParams(
            dimension_semantics=("arbitrary",),
            vmem_limit_bytes=64 * 1024 * 1024,
        ),
    )(xt, w1, b1, cw2, cb2, cwo, cbo, ww2, wb2, wwo, wbo)

    cls = cls_t.reshape(B, 80, H, W)
    wh = wh_t.reshape(B, 4, H, W)
    return (cls, wh)
